# baseline (speedup 1.0000x reference)
import jax
import jax.numpy as jnp
from jax import lax
from jax.experimental import pallas as pl
from jax.experimental.pallas import tpu as pltpu
from jax.experimental.pallas import tpu_sc as plsc

K = 19
NC, NS, L = 2, 16, 16
NW = NC * NS
CPT = 8
KC = K * CPT
CS = 8192


def kernel(features, labels):
    B, C, H, W = features.shape
    N = H * W
    feats3 = features.reshape(B, C, N)
    labels2 = labels.reshape(B, N)

    def sc_body(feats_hbm, labels_hbm, out_hbm, cnt_hbm,
                lab_v, feats_v, acc_v, cacc_v, sums_v, csums_v, mean_v):
        wid = lax.axis_index("s") * NC + lax.axis_index("c")
        c0 = wid * CPT
        iota = lax.iota(jnp.int32, L)
        ones = jnp.ones((L,), jnp.float32)
        zeros = jnp.zeros((L,), jnp.float32)

        def _z(i, c):
            acc_v[pl.ds(i * L, L)] = zeros
            return c
        lax.fori_loop(0, KC, _z, 0)

        def _zc(i, c):
            cacc_v[pl.ds(i * L, L)] = zeros
            return c
        lax.fori_loop(0, K, _zc, 0)

        for b in range(B):
            for ch in range(N // CS):
                pltpu.sync_copy(labels_hbm.at[b, pl.ds(ch * CS, CS)], lab_v)
                pltpu.sync_copy(
                    feats_hbm.at[b, pl.ds(c0, CPT), pl.ds(ch * CS, CS)],
                    feats_v)

                def _grp(g, c):
                    p = g * L
                    labs = lab_v[pl.ds(p, L)]
                    base = labs * (CPT * L) + iota
                    plsc.addupdate_scatter(cacc_v, [labs * L + iota], ones)
                    for cl in range(CPT):
                        f = feats_v[cl, pl.ds(p, L)]
                        plsc.addupdate_scatter(acc_v, [base + cl * L], f)
                    return c
                lax.fori_loop(0, CS // L, _grp, 0)

        lane_last = iota == (L - 1)

        def _fold(i, c):
            tot = plsc.cumsum(acc_v[pl.ds(i * L, L)])
            plsc.store_scatter(sums_v, [iota * 0 + i], tot, mask=lane_last)
            return c
        lax.fori_loop(0, KC, _fold, 0)

        def _foldc(i, c):
            tot = plsc.cumsum(cacc_v[pl.ds(i * L, L)])
            plsc.store_scatter(csums_v, [iota * 0 + i], tot, mask=lane_last)
            return c
        lax.fori_loop(0, K, _foldc, 0)

        for s in range(KC // L + 1):
            i = s * L + iota
            k_idx = lax.shift_right_logical(i, 3)
            cl_idx = jnp.bitwise_and(i, CPT - 1)
            valid = k_idx < K
            cnt = plsc.load_gather(csums_v, [jnp.minimum(k_idx, K - 1)])
            a = sums_v[pl.ds(s * L, L)]
            m = jnp.where(cnt > 0.0, a / jnp.maximum(cnt, 1.0), 0.0)
            plsc.store_scatter(mean_v, [cl_idx, k_idx], m, mask=valid)

        pltpu.sync_copy(mean_v, out_hbm.at[pl.ds(c0, CPT), :])

        @pl.when(wid == 0)
        def _wcnt():
            pltpu.sync_copy(csums_v, cnt_hbm)

    mesh = plsc.VectorSubcoreMesh(core_axis_name="c", subcore_axis_name="s",
                                  num_cores=NC, num_subcores=NS)
    protos, counts = pl.kernel(
        sc_body,
        out_type=[
            jax.ShapeDtypeStruct((C, K), jnp.float32),
            jax.ShapeDtypeStruct((2 * L,), jnp.float32),
        ],
        mesh=mesh,
        compiler_params=pltpu.CompilerParams(needs_layout_passes=False),
        scratch_types=[
            pltpu.VMEM((CS,), jnp.int32),
            pltpu.VMEM((CPT, CS), jnp.float32),
            pltpu.VMEM((KC * L,), jnp.float32),
            pltpu.VMEM((K * L,), jnp.float32),
            pltpu.VMEM((KC + L,), jnp.float32),
            pltpu.VMEM((2 * L,), jnp.float32),
            pltpu.VMEM((CPT, K), jnp.float32),
        ],
    )(feats3, labels2)

    return protos.T, counts[:K]


# speedup vs baseline: 1.9747x; 1.9747x over previous
import jax
import jax.numpy as jnp
from jax import lax
from jax.experimental import pallas as pl
from jax.experimental.pallas import tpu as pltpu
from jax.experimental.pallas import tpu_sc as plsc

K = 19
NC, NS, L = 2, 16, 16
NW = NC * NS
CPT = 8
KC = K * CPT
CS = 4096


def kernel(features, labels):
    B, C, H, W = features.shape
    N = H * W
    feats3 = features.reshape(B, C, N)
    labels2 = labels.reshape(B, N)
    cpb = N // CS
    nch = B * cpb

    def sc_body(feats_hbm, labels_hbm, out_hbm, cnt_hbm,
                lab_v, feats_v, acc_v, cacc_v, sums_v, csums_v, mean_v,
                sem_f0, sem_f1, sem_l0, sem_l1):
        sem_f = (sem_f0, sem_f1)
        sem_l = (sem_l0, sem_l1)
        wid = lax.axis_index("s") * NC + lax.axis_index("c")
        c0 = wid * CPT
        iota = lax.iota(jnp.int32, L)
        ones = jnp.ones((L,), jnp.float32)
        zeros = jnp.zeros((L,), jnp.float32)

        def chunk_src(c):
            b = lax.div(c, cpb)
            off = lax.rem(c, cpb) * CS
            return (feats_hbm.at[b, pl.ds(c0, CPT), pl.ds(off, CS)],
                    labels_hbm.at[b, pl.ds(off, CS)])

        def _z(i, c):
            acc_v[pl.ds(i * L, L)] = zeros
            return c
        lax.fori_loop(0, KC, _z, 0)

        def _zc(i, c):
            cacc_v[pl.ds(i * L, L)] = zeros
            return c
        lax.fori_loop(0, K, _zc, 0)

        for p in range(2):
            fsrc, lsrc = chunk_src(jnp.int32(p))
            pltpu.async_copy(fsrc, feats_v.at[p], sem_f[p])
            pltpu.async_copy(lsrc, lab_v.at[p], sem_l[p])

        def _outer(t, cr):
            for p in range(2):
                c = t * 2 + p
                fsrc, lsrc = chunk_src(c)
                pltpu.make_async_copy(fsrc, feats_v.at[p], sem_f[p]).wait()
                pltpu.make_async_copy(lsrc, lab_v.at[p], sem_l[p]).wait()

                @plsc.parallel_loop(0, CS // L, unroll=4)
                def _grp(g):
                    pp = g * L
                    labs = lab_v[p, pl.ds(pp, L)]
                    base = labs * (CPT * L) + iota
                    plsc.addupdate_scatter(cacc_v, [labs * L + iota], ones)
                    for cl in range(CPT):
                        f = feats_v[p, cl, pl.ds(pp, L)]
                        plsc.addupdate_scatter(acc_v, [base + cl * L], f)

                @pl.when(c + 2 < nch)
                def _prefetch():
                    fsrc2, lsrc2 = chunk_src(c + 2)
                    pltpu.async_copy(fsrc2, feats_v.at[p], sem_f[p])
                    pltpu.async_copy(lsrc2, lab_v.at[p], sem_l[p])
            return cr
        lax.fori_loop(0, nch // 2, _outer, 0)

        lane_last = iota == (L - 1)

        def _fold(i, c):
            tot = plsc.cumsum(acc_v[pl.ds(i * L, L)])
            plsc.store_scatter(sums_v, [iota * 0 + i], tot, mask=lane_last)
            return c
        lax.fori_loop(0, KC, _fold, 0)

        def _foldc(i, c):
            tot = plsc.cumsum(cacc_v[pl.ds(i * L, L)])
            plsc.store_scatter(csums_v, [iota * 0 + i], tot, mask=lane_last)
            return c
        lax.fori_loop(0, K, _foldc, 0)

        for s in range(KC // L + 1):
            i = s * L + iota
            k_idx = lax.shift_right_logical(i, 3)
            cl_idx = jnp.bitwise_and(i, CPT - 1)
            valid = k_idx < K
            cnt = plsc.load_gather(csums_v, [jnp.minimum(k_idx, K - 1)])
            a = sums_v[pl.ds(s * L, L)]
            m = jnp.where(cnt > 0.0, a / jnp.maximum(cnt, 1.0), 0.0)
            plsc.store_scatter(mean_v, [cl_idx, k_idx], m, mask=valid)

        pltpu.sync_copy(mean_v, out_hbm.at[pl.ds(c0, CPT), :])

        @pl.when(wid == 0)
        def _wcnt():
            pltpu.sync_copy(csums_v, cnt_hbm)

    mesh = plsc.VectorSubcoreMesh(core_axis_name="c", subcore_axis_name="s",
                                  num_cores=NC, num_subcores=NS)
    protos, counts = pl.kernel(
        sc_body,
        out_type=[
            jax.ShapeDtypeStruct((C, K), jnp.float32),
            jax.ShapeDtypeStruct((2 * L,), jnp.float32),
        ],
        mesh=mesh,
        compiler_params=pltpu.CompilerParams(needs_layout_passes=False),
        scratch_types=[
            pltpu.VMEM((2, CS), jnp.int32),
            pltpu.VMEM((2, CPT, CS), jnp.float32),
            pltpu.VMEM((KC * L,), jnp.float32),
            pltpu.VMEM((K * L,), jnp.float32),
            pltpu.VMEM((KC + L,), jnp.float32),
            pltpu.VMEM((2 * L,), jnp.float32),
            pltpu.VMEM((CPT, K), jnp.float32),
            pltpu.SemaphoreType.DMA,
            pltpu.SemaphoreType.DMA,
            pltpu.SemaphoreType.DMA,
            pltpu.SemaphoreType.DMA,
        ],
    )(feats3, labels2)

    return protos.T, counts[:K]
